# baseline (speedup 1.0000x reference)
import functools

import jax
import jax.numpy as jnp
from jax import lax
from jax.experimental import pallas as pl
from jax.experimental.pallas import tpu as pltpu
from jax.experimental.pallas import tpu_sc as plsc

NUM_CLASSES = 1000
HALF = 64
OUT_DIM = 2 * HALF
B, N = 4096, 200
TOT = B * N

NC, NS = 2, 16
NW = NC * NS
PER_W = TOT // NW
CH = 128
K = 2
GRP = K * CH
N_GRP = PER_W // GRP


def _sc_gather(ids_flat, table_pad):
    mesh = plsc.VectorSubcoreMesh(core_axis_name="c", subcore_axis_name="s")

    @functools.partial(
        pl.kernel,
        out_type=jax.ShapeDtypeStruct((TOT, OUT_DIM), jnp.float32),
        mesh=mesh,
        compiler_params=pltpu.CompilerParams(use_tc_tiling_on_sc=True),
        scratch_types=[
            pltpu.VMEM((PER_W,), jnp.int32),
            pltpu.VMEM((GRP, OUT_DIM), jnp.float32),
            pltpu.VMEM((GRP, OUT_DIM), jnp.float32),
            pltpu.VMEM_SHARED((NUM_CLASSES, OUT_DIM), jnp.float32),
            pltpu.SemaphoreType.DMA,
            pltpu.SemaphoreType.DMA,
            pltpu.SemaphoreType.DMA,
            pltpu.SemaphoreType.DMA,
        ],
    )
    def sc_body(ids_hbm, table_hbm, out_hbm, idx_v, rows0, rows1, tab_s,
                g0, g1, w0, w1):
        cid = lax.axis_index("c")
        sid = lax.axis_index("s")
        wid = sid * NC + cid
        base = wid * PER_W

        @pl.when(sid == 0)
        def _stage_table():
            pltpu.sync_copy(table_hbm, tab_s)

        plsc.subcore_barrier()
        pltpu.sync_copy(ids_hbm.at[pl.ds(base, PER_W)], idx_v)

        def issue_gathers(g, rows, gsem):
            for j in range(K):
                pltpu.async_copy(
                    tab_s.at[idx_v.at[pl.ds(g * GRP + j * CH, CH)]],
                    rows.at[pl.ds(j * CH, CH)],
                    gsem,
                )

        def drain_gathers(rows, gsem):
            pltpu.make_async_copy(tab_s.at[pl.ds(0, GRP)], rows, gsem).wait()

        def issue_write(g, rows, wsem):
            pltpu.async_copy(rows, out_hbm.at[pl.ds(base + g * GRP, GRP)], wsem)

        def drain_write(rows, wsem):
            pltpu.make_async_copy(rows, out_hbm.at[pl.ds(0, GRP)], wsem).wait()

        issue_gathers(0, rows0, g0)
        issue_gathers(1, rows1, g1)

        @pl.loop(0, N_GRP, step=2)
        def _grp(g):
            drain_gathers(rows0, g0)
            issue_write(g, rows0, w0)
            drain_gathers(rows1, g1)
            issue_write(g + 1, rows1, w1)

            @pl.when(g + 2 < N_GRP)
            def _refill0():
                drain_write(rows0, w0)
                issue_gathers(g + 2, rows0, g0)

            @pl.when(g + 3 < N_GRP)
            def _refill1():
                drain_write(rows1, w1)
                issue_gathers(g + 3, rows1, g1)

        drain_write(rows0, w0)
        drain_write(rows1, w1)

    return sc_body(ids_flat, table_pad)


BLK = 8192


def _tc_body(gath_ref, coords_ref, w1_ref, b1_ref, w2_ref, b2_ref, out_ref):
    c = coords_ref[...]
    h = (
        jax.lax.dot_general(
            c, w1_ref[...], (((1,), (0,)), ((), ())),
            preferred_element_type=jnp.float32,
        )
        + b1_ref[...]
    )
    h = jnp.maximum(h, 0.0)
    coord_emb = (
        jax.lax.dot_general(
            h, w2_ref[...], (((1,), (0,)), ((), ())),
            preferred_element_type=jnp.float32,
        )
        + b2_ref[...]
    )
    out_ref[...] = jnp.concatenate([gath_ref[:, :HALF], coord_emb], axis=1)


def _tc_mlp(sc_out, coords_flat, W1, b1, W2, b2):
    grid = (TOT // BLK,)
    return pl.pallas_call(
        _tc_body,
        grid=grid,
        in_specs=[
            pl.BlockSpec((BLK, OUT_DIM), lambda i: (i, 0)),
            pl.BlockSpec((BLK, 3), lambda i: (i, 0)),
            pl.BlockSpec((3, HALF), lambda i: (0, 0)),
            pl.BlockSpec((1, HALF), lambda i: (0, 0)),
            pl.BlockSpec((HALF, HALF), lambda i: (0, 0)),
            pl.BlockSpec((1, HALF), lambda i: (0, 0)),
        ],
        out_specs=pl.BlockSpec((BLK, OUT_DIM), lambda i: (i, 0)),
        out_shape=jax.ShapeDtypeStruct((TOT, OUT_DIM), jnp.float32),
        input_output_aliases={0: 0},
    )(sc_out, coords_flat, W1, b1, W2, b2)


def kernel(class_ids, coords, emb_table, W1, b1, W2, b2):
    ids_flat = class_ids.reshape(TOT).astype(jnp.int32)
    coords_flat = coords.reshape(TOT, 3)
    table_pad = jnp.pad(emb_table, ((0, 0), (0, OUT_DIM - HALF)))
    sc_out = _sc_gather(ids_flat, table_pad)
    out = _tc_mlp(
        sc_out, coords_flat, W1, b1.reshape(1, HALF), W2, b2.reshape(1, HALF)
    )
    return out.reshape(B, N, OUT_DIM)


# speedup vs baseline: 4.5682x; 4.5682x over previous
import functools

import jax
import jax.numpy as jnp
from jax import lax
from jax.experimental import pallas as pl
from jax.experimental.pallas import tpu as pltpu
from jax.experimental.pallas import tpu_sc as plsc

NUM_CLASSES = 1000
HALF = 64
OUT_DIM = 2 * HALF
B, N = 4096, 200
TOT = B * N

NC, NS = 2, 16
NW = NC * NS
PER_W = TOT // NW
CH = 128
K = 2
GRP = K * CH
N_GRP = PER_W // GRP


def _sc_gather(ids_flat, table_pad):
    mesh = plsc.VectorSubcoreMesh(core_axis_name="c", subcore_axis_name="s")

    @functools.partial(
        pl.kernel,
        out_type=jax.ShapeDtypeStruct((TOT, OUT_DIM), jnp.float32),
        mesh=mesh,
        compiler_params=pltpu.CompilerParams(use_tc_tiling_on_sc=True),
        scratch_types=[
            pltpu.VMEM((PER_W,), jnp.int32),
            pltpu.VMEM((GRP, OUT_DIM), jnp.float32),
            pltpu.VMEM((GRP, OUT_DIM), jnp.float32),
            pltpu.VMEM_SHARED((NUM_CLASSES, OUT_DIM), jnp.float32),
            pltpu.SemaphoreType.DMA,
            pltpu.SemaphoreType.DMA,
            pltpu.SemaphoreType.DMA,
            pltpu.SemaphoreType.DMA,
        ],
    )
    def sc_body(ids_hbm, table_hbm, out_hbm, idx_v, rows0, rows1, tab_s,
                g0, g1, w0, w1):
        cid = lax.axis_index("c")
        sid = lax.axis_index("s")
        wid = sid * NC + cid
        base = wid * PER_W

        @pl.when(sid == 0)
        def _stage_table():
            pltpu.sync_copy(table_hbm, tab_s)

        plsc.subcore_barrier()
        pltpu.sync_copy(ids_hbm.at[pl.ds(base, PER_W)], idx_v)

        def issue_gathers(g, rows, gsem):
            for j in range(K):
                pltpu.async_copy(
                    tab_s.at[idx_v.at[pl.ds(g * GRP + j * CH, CH)]],
                    rows.at[pl.ds(j * CH, CH)],
                    gsem,
                )

        def drain_gathers(rows, gsem):
            pltpu.make_async_copy(tab_s.at[pl.ds(0, GRP)], rows, gsem).wait()

        def issue_write(g, rows, wsem):
            pltpu.async_copy(rows, out_hbm.at[pl.ds(base + g * GRP, GRP)], wsem)

        def drain_write(rows, wsem):
            pltpu.make_async_copy(rows, out_hbm.at[pl.ds(0, GRP)], wsem).wait()

        issue_gathers(0, rows0, g0)
        issue_gathers(1, rows1, g1)

        @pl.loop(0, N_GRP, step=2)
        def _grp(g):
            drain_gathers(rows0, g0)
            issue_write(g, rows0, w0)
            drain_gathers(rows1, g1)
            issue_write(g + 1, rows1, w1)

            @pl.when(g + 2 < N_GRP)
            def _refill0():
                drain_write(rows0, w0)
                issue_gathers(g + 2, rows0, g0)

            @pl.when(g + 3 < N_GRP)
            def _refill1():
                drain_write(rows1, w1)
                issue_gathers(g + 3, rows1, g1)

        drain_write(rows0, w0)
        drain_write(rows1, w1)

    return sc_body(ids_flat, table_pad)


BB = 128
NB = 40
TB = BB * NB


def _tc_body(gath_ref, c3_ref, w1_ref, b1_ref, w2_ref, b2_ref, out_ref):
    c = c3_ref[...]
    ct = jnp.transpose(c, (0, 2, 1))
    lhs = ct.reshape(3, TB)
    h = (
        jax.lax.dot_general(
            lhs, w1_ref[...], (((0,), (0,)), ((), ())),
            preferred_element_type=jnp.float32,
        )
        + b1_ref[...]
    )
    h = jnp.maximum(h, 0.0)
    coord_emb = (
        jax.lax.dot_general(
            h, w2_ref[...], (((1,), (0,)), ((), ())),
            preferred_element_type=jnp.float32,
        )
        + b2_ref[...]
    )
    g = gath_ref[...].reshape(TB, OUT_DIM)
    out = jnp.concatenate([g[:, :HALF], coord_emb], axis=1)
    out_ref[...] = out.reshape(BB, NB, OUT_DIM)


def _tc_mlp(sc_out3, coords_t, W1, b1, W2, b2):
    grid = (B // BB, N // NB)
    return pl.pallas_call(
        _tc_body,
        grid=grid,
        in_specs=[
            pl.BlockSpec((BB, NB, OUT_DIM), lambda i, j: (i, j, 0)),
            pl.BlockSpec((3, NB, BB), lambda i, j: (0, j, i)),
            pl.BlockSpec((3, HALF), lambda i, j: (0, 0)),
            pl.BlockSpec((1, HALF), lambda i, j: (0, 0)),
            pl.BlockSpec((HALF, HALF), lambda i, j: (0, 0)),
            pl.BlockSpec((1, HALF), lambda i, j: (0, 0)),
        ],
        out_specs=pl.BlockSpec((BB, NB, OUT_DIM), lambda i, j: (i, j, 0)),
        out_shape=jax.ShapeDtypeStruct((B, N, OUT_DIM), jnp.float32),
        input_output_aliases={0: 0},
    )(sc_out3, coords_t, W1, b1, W2, b2)


def kernel(class_ids, coords, emb_table, W1, b1, W2, b2):
    ids_flat = class_ids.reshape(TOT).astype(jnp.int32)
    coords_t = jnp.transpose(coords, (2, 1, 0))
    table_pad = jnp.pad(emb_table, ((0, 0), (0, OUT_DIM - HALF)))
    sc_out = _sc_gather(ids_flat, table_pad)
    return _tc_mlp(
        sc_out.reshape(B, N, OUT_DIM), coords_t,
        W1, b1.reshape(1, HALF), W2, b2.reshape(1, HALF),
    )
